# baseline (speedup 1.0000x reference)
import math

import jax
import jax.numpy as jnp
from jax import lax
from jax.experimental import pallas as pl
from jax.experimental.pallas import tpu as pltpu
from jax.experimental.pallas import tpu_sc as plsc

VOCAB = 100000
N_EMBD = 64
BUCKET = 8191
NUM_TABLES = 3
N_TOKENS = 1024 * 200

NUM_CORES = 2
NUM_SUBCORES = 16
LANES = 16
NW = NUM_CORES * NUM_SUBCORES
TOK_PER_W = N_TOKENS // NW
CHUNK = 128
NCHUNK = TOK_PER_W // CHUNK
META = NUM_TABLES * CHUNK
SCALE = math.sqrt(N_EMBD)


def _splat(v):
    return jnp.full((LANES,), v, jnp.int32)


def _make_lookup():
    mesh = plsc.VectorSubcoreMesh(core_axis_name="c", subcore_axis_name="s")

    def body(x_hbm, ai_hbm, imp_hbm, tab_hbm, out_hbm,
             x_v, gidx_v, ai_v, w_v, idx_v, rows_v, out_v,
             meta_sem, rows_sem, out_sem):
        wid = lax.axis_index("s") * NUM_CORES + lax.axis_index("c")
        lane_iota = lax.iota(jnp.int32, LANES)

        def stage_a(k, p):
            base = wid * TOK_PER_W + k * CHUNK
            pltpu.sync_copy(x_hbm.at[pl.ds(base, CHUNK)], x_v)
            for g in range(CHUNK // LANES):
                xv = x_v[pl.ds(g * LANES, LANES)]
                tvec = lane_iota + (g * LANES)
                for i in range(NUM_TABLES):
                    plsc.store_scatter(gidx_v,
                                       [p * META + NUM_TABLES * tvec + i],
                                       NUM_TABLES * xv + i)
            hs = []
            for r in range(NUM_TABLES):
                seg = gidx_v.at[pl.ds(p * META + r * CHUNK, CHUNK)]
                hs.append(pltpu.async_copy(
                    ai_hbm.at[seg],
                    ai_v.at[pl.ds(p * META + r * CHUNK, CHUNK)], meta_sem))
                hs.append(pltpu.async_copy(
                    imp_hbm.at[seg],
                    w_v.at[pl.ds(p * META + r * CHUNK, CHUNK)], meta_sem))
            for h in hs:
                h.wait()
            for i in range(NUM_TABLES):
                for g in range(CHUNK // LANES):
                    tvec = lane_iota + (g * LANES)
                    vals = plsc.load_gather(
                        ai_v, [p * META + NUM_TABLES * tvec + i]) + i * BUCKET
                    idx_v[p, i, pl.ds(g * LANES, LANES)] = vals
            for i in range(NUM_TABLES):
                pltpu.async_copy(tab_hbm.at[idx_v.at[p, i]], rows_v.at[p, i],
                                 rows_sem.at[p])

        def wait_rows(p):
            for i in range(NUM_TABLES):
                pltpu.make_async_copy(tab_hbm.at[idx_v.at[p, i]], rows_v.at[p, i],
                                      rows_sem.at[p]).wait()

        def drain_out(p):
            pltpu.make_async_copy(out_v.at[p], out_hbm.at[pl.ds(0, CHUNK)],
                                  out_sem.at[p]).wait()

        def compute(k, p):
            def tok_body(tt, carry2):
                for u in range(2):
                    t = tt * 2 + u
                    wbase = p * META + NUM_TABLES * t
                    w0 = plsc.load_gather(w_v, [_splat(wbase)]) * SCALE
                    w1 = plsc.load_gather(w_v, [_splat(wbase + 1)]) * SCALE
                    w2 = plsc.load_gather(w_v, [_splat(wbase + 2)]) * SCALE
                    for q in range(N_EMBD // LANES):
                        sl = pl.ds(q * LANES, LANES)
                        acc = (w0 * rows_v[p, 0, t, sl]
                               + w1 * rows_v[p, 1, t, sl]
                               + w2 * rows_v[p, 2, t, sl])
                        out_v[p, t, sl] = acc
                return carry2

            lax.fori_loop(0, CHUNK // 2, tok_body, 0)
            base = wid * TOK_PER_W + k * CHUNK
            pltpu.async_copy(out_v.at[p], out_hbm.at[pl.ds(base, CHUNK)],
                             out_sem.at[p])

        stage_a(0, 0)

        def chunk_pair(kk, carry):
            for p in range(2):
                k = kk * 2 + p

                @pl.when(k + 1 < NCHUNK)
                def _():
                    stage_a(k + 1, 1 - p)

                wait_rows(p)

                @pl.when(k >= 2)
                def _():
                    drain_out(p)

                compute(k, p)
            return carry

        lax.fori_loop(0, NCHUNK // 2, chunk_pair, 0)
        drain_out(0)
        drain_out(1)

    return pl.kernel(
        body,
        out_type=jax.ShapeDtypeStruct((N_TOKENS, N_EMBD), jnp.float32),
        mesh=mesh,
        compiler_params=pltpu.CompilerParams(
            needs_layout_passes=False, use_tc_tiling_on_sc=False),
        scratch_types=[
            pltpu.VMEM((CHUNK,), jnp.int32),
            pltpu.VMEM((2 * META,), jnp.int32),
            pltpu.VMEM((2 * META,), jnp.int32),
            pltpu.VMEM((2 * META,), jnp.float32),
            pltpu.VMEM((2, NUM_TABLES, CHUNK), jnp.int32),
            pltpu.VMEM((2, NUM_TABLES, CHUNK, N_EMBD), jnp.float32),
            pltpu.VMEM((2, CHUNK, N_EMBD), jnp.float32),
            pltpu.SemaphoreType.DMA,
            pltpu.SemaphoreType.DMA((2,)),
            pltpu.SemaphoreType.DMA((2,)),
        ],
    )


def kernel(x, all_indices, tables, importance):
    x_flat = x.reshape(-1)
    ai_flat = all_indices.reshape(-1)
    imp_flat = importance.reshape(-1)
    tab = tables.reshape(NUM_TABLES * BUCKET, N_EMBD)
    lookup = _make_lookup()
    out = lookup(x_flat, ai_flat, imp_flat, tab)
    return out.reshape(x.shape + (N_EMBD,))
